# speedup vs baseline: 1.8975x; 1.8975x over previous
import functools

import jax
import jax.numpy as jnp
from jax.experimental import pallas as pl
from jax.experimental.pallas import tpu as pltpu

E = 16
K = 2
CAP_F = 1.25
AUX_W = 0.01

_BT = 512


def _gating_body(x_ref, wg_ref, p_ref, g2_ref, e0_ref, e1_ref, oh0_ref, oh1_ref,
                 me_ref, ce_ref):
    logits = jnp.dot(x_ref[...], wg_ref[...], preferred_element_type=jnp.float32)
    m = jnp.max(logits, axis=1, keepdims=True)
    ex = jnp.exp(logits - m)
    gates = ex / jnp.sum(ex, axis=1, keepdims=True)
    iota = jax.lax.broadcasted_iota(jnp.int32, gates.shape, 1)
    m1 = jnp.max(gates, axis=1, keepdims=True)
    is1 = gates == m1
    i1 = jnp.min(jnp.where(is1, iota, E), axis=1, keepdims=True)
    oh1h = (iota == i1)
    gm = jnp.where(oh1h, -jnp.inf, gates)
    m2 = jnp.max(gm, axis=1, keepdims=True)
    i2 = jnp.min(jnp.where(gm == m2, iota, E), axis=1, keepdims=True)
    oh2h = (iota == i2)
    p_ref[0, 0, :] = m1[:, 0]
    g2_ref[0, 0, :] = m2[:, 0]
    e0_ref[0, 0, :] = i1[:, 0]
    e1_ref[0, 0, :] = i2[:, 0]
    oh0_ref[...] = oh1h.astype(jnp.float32)
    oh1_ref[...] = oh2h.astype(jnp.float32)
    me_ref[0, 0, :] = jnp.sum(gates, axis=0)
    ce_ref[0, 0, :] = jnp.sum(oh1h.astype(jnp.float32), axis=0)


def _routing_body(p_ref, g2_ref, e0_ref, e1_ref, oh0_ref, oh1_ref, me_ref, ce_ref,
                  sslot0_ref, sslot1_ref, gslot0_ref, gslot1_ref, w0_ref, w1_ref,
                  laux_ref, *, n_tokens, n_blocks, capacity, trash_row):
    i = pl.program_id(0)
    base_i = i * _BT
    pi = p_ref[0, pl.ds(base_i, _BT)].reshape(_BT, 1)
    ig = jax.lax.broadcasted_iota(jnp.int32, (_BT, 1), 0) + base_i

    def jstep(j, accs):
        acc0, acc1 = accs
        base_j = j * _BT
        pj = p_ref[0, pl.ds(base_j, _BT)].reshape(1, _BT)
        jg = jax.lax.broadcasted_iota(jnp.int32, (1, _BT), 1) + base_j
        cmp = ((pj > pi) | ((pj == pi) & (jg < ig))).astype(jnp.float32)
        oh0j = oh0_ref[pl.ds(base_j, _BT), :]
        oh1j = oh1_ref[pl.ds(base_j, _BT), :]
        acc0 = acc0 + jnp.dot(cmp, oh0j, preferred_element_type=jnp.float32)
        acc1 = acc1 + jnp.dot(cmp, oh1j, preferred_element_type=jnp.float32)
        return acc0, acc1

    acc0, acc1 = jax.lax.fori_loop(
        0, n_blocks, jstep,
        (jnp.zeros((_BT, E), jnp.float32), jnp.zeros((_BT, E), jnp.float32)))

    oh0i = oh0_ref[pl.ds(base_i, _BT), :]
    oh1i = oh1_ref[pl.ds(base_i, _BT), :]
    rank0 = jnp.sum(acc0 * oh0i, axis=1)
    rank1 = jnp.sum(acc1 * oh1i, axis=1)
    cnt0 = jnp.sum(oh0_ref[...], axis=0, keepdims=True)
    off1 = jnp.sum(cnt0 * oh1i, axis=1)
    pos0 = rank0.astype(jnp.int32)
    pos1 = (rank1 + off1).astype(jnp.int32)
    e0 = e0_ref[0, pl.ds(base_i, _BT)]
    e1 = e1_ref[0, pl.ds(base_i, _BT)]
    v0 = pos0 < capacity
    v1 = pos1 < capacity
    slot0 = e0 * capacity + jnp.minimum(pos0, capacity - 1)
    slot1 = e1 * capacity + jnp.minimum(pos1, capacity - 1)
    sslot0_ref[0, 0, :] = jnp.where(v0, slot0, trash_row)
    sslot1_ref[0, 0, :] = jnp.where(v1, slot1, trash_row)
    gslot0_ref[0, 0, :] = slot0
    gslot1_ref[0, 0, :] = slot1
    w0_ref[0, 0, :] = jnp.where(v0, p_ref[0, pl.ds(base_i, _BT)], 0.0)
    w1_ref[0, 0, :] = jnp.where(v1, g2_ref[0, pl.ds(base_i, _BT)], 0.0)

    @pl.when(i == 0)
    def _():
        me = jnp.sum(me_ref[:, 0, :], axis=0) / n_tokens
        ce = jnp.sum(ce_ref[:, 0, :], axis=0) / n_tokens
        laux = jnp.sum(me * ce) * E * AUX_W
        laux_ref[...] = jnp.full((8, 128), laux, jnp.float32)


def _ffn_body(x_ref, w1_ref, b1_ref, w2_ref, b2_ref, out_ref):
    e_id = pl.program_id(0)
    h_id = pl.program_id(1)
    xb = x_ref[...].astype(jnp.bfloat16)
    w1b = w1_ref[0].astype(jnp.bfloat16)
    b1v = b1_ref[pl.ds(e_id, 1), pl.ds(h_id * 1024, 1024)]
    h = jnp.dot(xb, w1b, preferred_element_type=jnp.float32) + b1v
    hg = 0.5 * h * (1.0 + jax.lax.erf(h * 0.7071067811865476))
    hb = hg.astype(jnp.bfloat16)
    w2b = w2_ref[0].astype(jnp.bfloat16)
    part = jnp.dot(hb, w2b, preferred_element_type=jnp.float32)

    @pl.when(h_id == 0)
    def _():
        out_ref[...] = part + b2_ref[pl.ds(e_id, 1), :]

    @pl.when(h_id != 0)
    def _():
        out_ref[...] = out_ref[...] + part


def _run_gating(xf, wg, n_blocks):
    n, d = xf.shape
    tok3 = lambda dt: jax.ShapeDtypeStruct((n_blocks, 1, _BT), dt)
    out_shapes = (
        tok3(jnp.float32), tok3(jnp.float32),
        tok3(jnp.int32), tok3(jnp.int32),
        jax.ShapeDtypeStruct((n, E), jnp.float32),
        jax.ShapeDtypeStruct((n, E), jnp.float32),
        jax.ShapeDtypeStruct((n_blocks, 1, E), jnp.float32),
        jax.ShapeDtypeStruct((n_blocks, 1, E), jnp.float32),
    )
    blk3 = lambda: pl.BlockSpec((1, 1, _BT), lambda i: (i, 0, 0))
    return pl.pallas_call(
        _gating_body,
        grid=(n_blocks,),
        in_specs=[
            pl.BlockSpec((_BT, d), lambda i: (i, 0)),
            pl.BlockSpec(wg.shape, lambda i: (0, 0)),
        ],
        out_specs=(
            blk3(), blk3(), blk3(), blk3(),
            pl.BlockSpec((_BT, E), lambda i: (i, 0)),
            pl.BlockSpec((_BT, E), lambda i: (i, 0)),
            pl.BlockSpec((1, 1, E), lambda i: (i, 0, 0)),
            pl.BlockSpec((1, 1, E), lambda i: (i, 0, 0)),
        ),
        out_shape=out_shapes,
    )(xf, wg)


def _run_routing(p, g2, e0, e1, oh0, oh1, me_p, ce_p, n_tokens, n_blocks,
                 capacity, trash_row):
    tok3 = lambda dt: jax.ShapeDtypeStruct((n_blocks, 1, _BT), dt)
    out_shapes = (
        tok3(jnp.int32), tok3(jnp.int32), tok3(jnp.int32), tok3(jnp.int32),
        tok3(jnp.float32), tok3(jnp.float32),
        jax.ShapeDtypeStruct((8, 128), jnp.float32),
    )
    blk3 = lambda: pl.BlockSpec((1, 1, _BT), lambda i: (i, 0, 0))
    full = lambda shape: pl.BlockSpec(shape, lambda i: tuple(0 for _ in shape))
    body = functools.partial(
        _routing_body, n_tokens=n_tokens, n_blocks=n_blocks,
        capacity=capacity, trash_row=trash_row)
    return pl.pallas_call(
        body,
        grid=(n_blocks,),
        in_specs=[
            full((1, n_tokens)), full((1, n_tokens)),
            full((1, n_tokens)), full((1, n_tokens)),
            full((n_tokens, E)), full((n_tokens, E)),
            full((n_blocks, 1, E)), full((n_blocks, 1, E)),
        ],
        out_specs=(
            blk3(), blk3(), blk3(), blk3(), blk3(), blk3(),
            pl.BlockSpec((8, 128), lambda i: (0, 0)),
        ),
        out_shape=out_shapes,
    )(p, g2, e0, e1, oh0, oh1, me_p, ce_p)


def _run_ffn(buf, w1, b1, w2, b2, capacity, n_rows):
    d_model = w1.shape[1]
    d_ff = w1.shape[2]
    n_h = d_ff // 1024
    return pl.pallas_call(
        _ffn_body,
        grid=(E, n_h),
        in_specs=[
            pl.BlockSpec((capacity, d_model), lambda e, h: (e, 0)),
            pl.BlockSpec((1, d_model, 1024), lambda e, h: (e, 0, h)),
            pl.BlockSpec((E, d_ff), lambda e, h: (0, 0)),
            pl.BlockSpec((1, 1024, d_model), lambda e, h: (e, h, 0)),
            pl.BlockSpec((E, d_model), lambda e, h: (0, 0)),
        ],
        out_specs=pl.BlockSpec((capacity, d_model), lambda e, h: (e, 0)),
        out_shape=jax.ShapeDtypeStruct((n_rows, d_model), jnp.float32),
    )(buf, w1, b1, w2, b2)


def kernel(x, wg, w1, b1, w2, b2):
    B, S, D = x.shape
    N = B * S
    n_blocks = N // _BT
    capacity = int(CAP_F * K * N / E)
    n_rows = (E + 1) * capacity
    trash_row = E * capacity
    xf = x.reshape(N, D)

    p3, g23, e03, e13, oh0, oh1, me_p, ce_p = _run_gating(xf, wg, n_blocks)
    p = p3.reshape(1, N)
    g2 = g23.reshape(1, N)
    e0 = e03.reshape(1, N)
    e1 = e13.reshape(1, N)

    (ss03, ss13, gs03, gs13, w03, w13, laux) = _run_routing(
        p, g2, e0, e1, oh0, oh1, me_p, ce_p, N, n_blocks, capacity, trash_row)
    ss0 = ss03.reshape(N)
    ss1 = ss13.reshape(N)
    gs0 = gs03.reshape(N)
    gs1 = gs13.reshape(N)
    w0 = w03.reshape(N)
    w1v = w13.reshape(N)

    buf = jnp.zeros((n_rows, D), jnp.float32)
    buf = buf.at[ss0, :].set(xf)
    buf = buf.at[ss1, :].set(xf)

    out = _run_ffn(buf, w1, b1, w2, b2, capacity, n_rows)

    y = out[gs0] * w0[:, None] + out[gs1] * w1v[:, None]
    return y.reshape(B, S, D), laux[0, 0]


# speedup vs baseline: 2.5255x; 1.3309x over previous
import functools

import jax
import jax.numpy as jnp
from jax import lax
from jax.experimental import pallas as pl
from jax.experimental.pallas import tpu as pltpu
from jax.experimental.pallas import tpu_sc as plsc

_SC_CORES = 2
_SC_SUBCORES = 16
_NW = _SC_CORES * _SC_SUBCORES

E = 16
K = 2
CAP_F = 1.25
AUX_W = 0.01

_BT = 512


def _gating_body(x_ref, wg_ref, p_ref, g2_ref, e0_ref, e1_ref, oh0_ref, oh1_ref,
                 me_ref, ce_ref):
    logits = jnp.dot(x_ref[...], wg_ref[...], preferred_element_type=jnp.float32)
    m = jnp.max(logits, axis=1, keepdims=True)
    ex = jnp.exp(logits - m)
    gates = ex / jnp.sum(ex, axis=1, keepdims=True)
    iota = jax.lax.broadcasted_iota(jnp.int32, gates.shape, 1)
    m1 = jnp.max(gates, axis=1, keepdims=True)
    is1 = gates == m1
    i1 = jnp.min(jnp.where(is1, iota, E), axis=1, keepdims=True)
    oh1h = (iota == i1)
    gm = jnp.where(oh1h, -jnp.inf, gates)
    m2 = jnp.max(gm, axis=1, keepdims=True)
    i2 = jnp.min(jnp.where(gm == m2, iota, E), axis=1, keepdims=True)
    oh2h = (iota == i2)
    p_ref[0, 0, :] = m1[:, 0]
    g2_ref[0, 0, :] = m2[:, 0]
    e0_ref[0, 0, :] = i1[:, 0]
    e1_ref[0, 0, :] = i2[:, 0]
    oh0_ref[...] = oh1h.astype(jnp.float32)
    oh1_ref[...] = oh2h.astype(jnp.float32)
    me_ref[0, 0, :] = jnp.sum(gates, axis=0)
    ce_ref[0, 0, :] = jnp.sum(oh1h.astype(jnp.float32), axis=0)


def _routing_body(p_ref, g2_ref, e0_ref, e1_ref, oh0_ref, oh1_ref, me_ref, ce_ref,
                  sslot0_ref, sslot1_ref, gslot0_ref, gslot1_ref, w0_ref, w1_ref,
                  laux_ref, *, n_tokens, n_blocks, capacity, trash_row):
    i = pl.program_id(0)
    base_i = i * _BT
    pi = p_ref[0, pl.ds(base_i, _BT)].reshape(_BT, 1)
    ig = jax.lax.broadcasted_iota(jnp.int32, (_BT, 1), 0) + base_i

    def jstep(j, accs):
        acc0, acc1 = accs
        base_j = j * _BT
        pj = p_ref[0, pl.ds(base_j, _BT)].reshape(1, _BT)
        jg = jax.lax.broadcasted_iota(jnp.int32, (1, _BT), 1) + base_j
        cmp = ((pj > pi) | ((pj == pi) & (jg < ig))).astype(jnp.float32)
        oh0j = oh0_ref[pl.ds(base_j, _BT), :]
        oh1j = oh1_ref[pl.ds(base_j, _BT), :]
        acc0 = acc0 + jnp.dot(cmp, oh0j, preferred_element_type=jnp.float32)
        acc1 = acc1 + jnp.dot(cmp, oh1j, preferred_element_type=jnp.float32)
        return acc0, acc1

    acc0, acc1 = jax.lax.fori_loop(
        0, n_blocks, jstep,
        (jnp.zeros((_BT, E), jnp.float32), jnp.zeros((_BT, E), jnp.float32)))

    oh0i = oh0_ref[pl.ds(base_i, _BT), :]
    oh1i = oh1_ref[pl.ds(base_i, _BT), :]
    rank0 = jnp.sum(acc0 * oh0i, axis=1)
    rank1 = jnp.sum(acc1 * oh1i, axis=1)
    cnt0 = jnp.sum(oh0_ref[...], axis=0, keepdims=True)
    off1 = jnp.sum(cnt0 * oh1i, axis=1)
    pos0 = rank0.astype(jnp.int32)
    pos1 = (rank1 + off1).astype(jnp.int32)
    e0 = e0_ref[0, pl.ds(base_i, _BT)]
    e1 = e1_ref[0, pl.ds(base_i, _BT)]
    v0 = pos0 < capacity
    v1 = pos1 < capacity
    slot0 = e0 * capacity + jnp.minimum(pos0, capacity - 1)
    slot1 = e1 * capacity + jnp.minimum(pos1, capacity - 1)
    sslot0_ref[0, 0, :] = jnp.where(v0, slot0, trash_row)
    sslot1_ref[0, 0, :] = jnp.where(v1, slot1, trash_row)
    gslot0_ref[0, 0, :] = slot0
    gslot1_ref[0, 0, :] = slot1
    w0_ref[0, 0, :] = jnp.where(v0, p_ref[0, pl.ds(base_i, _BT)], 0.0)
    w1_ref[0, 0, :] = jnp.where(v1, g2_ref[0, pl.ds(base_i, _BT)], 0.0)

    @pl.when(i == 0)
    def _():
        me = jnp.sum(me_ref[:, 0, :], axis=0) / n_tokens
        ce = jnp.sum(ce_ref[:, 0, :], axis=0) / n_tokens
        laux = jnp.sum(me * ce) * E * AUX_W
        laux_ref[...] = jnp.full((8, 128), laux, jnp.float32)


def _ffn_body(x_ref, w1_ref, b1_ref, w2_ref, b2_ref, out_ref):
    e_id = pl.program_id(0)
    h_id = pl.program_id(1)
    xb = x_ref[...].astype(jnp.bfloat16)
    w1b = w1_ref[0].astype(jnp.bfloat16)
    b1v = b1_ref[pl.ds(e_id, 1), pl.ds(h_id * 1024, 1024)]
    h = jnp.dot(xb, w1b, preferred_element_type=jnp.float32) + b1v
    hg = 0.5 * h * (1.0 + jax.lax.erf(h * 0.7071067811865476))
    hb = hg.astype(jnp.bfloat16)
    w2b = w2_ref[0].astype(jnp.bfloat16)
    part = jnp.dot(hb, w2b, preferred_element_type=jnp.float32)

    @pl.when(h_id == 0)
    def _():
        out_ref[...] = part + b2_ref[pl.ds(e_id, 1), :]

    @pl.when(h_id != 0)
    def _():
        out_ref[...] = out_ref[...] + part


def _run_gating(xf, wg, n_blocks):
    n, d = xf.shape
    tok3 = lambda dt: jax.ShapeDtypeStruct((n_blocks, 1, _BT), dt)
    out_shapes = (
        tok3(jnp.float32), tok3(jnp.float32),
        tok3(jnp.int32), tok3(jnp.int32),
        jax.ShapeDtypeStruct((n, E), jnp.float32),
        jax.ShapeDtypeStruct((n, E), jnp.float32),
        jax.ShapeDtypeStruct((n_blocks, 1, E), jnp.float32),
        jax.ShapeDtypeStruct((n_blocks, 1, E), jnp.float32),
    )
    blk3 = lambda: pl.BlockSpec((1, 1, _BT), lambda i: (i, 0, 0))
    return pl.pallas_call(
        _gating_body,
        grid=(n_blocks,),
        in_specs=[
            pl.BlockSpec((_BT, d), lambda i: (i, 0)),
            pl.BlockSpec(wg.shape, lambda i: (0, 0)),
        ],
        out_specs=(
            blk3(), blk3(), blk3(), blk3(),
            pl.BlockSpec((_BT, E), lambda i: (i, 0)),
            pl.BlockSpec((_BT, E), lambda i: (i, 0)),
            pl.BlockSpec((1, 1, E), lambda i: (i, 0, 0)),
            pl.BlockSpec((1, 1, E), lambda i: (i, 0, 0)),
        ),
        out_shape=out_shapes,
    )(xf, wg)


def _run_routing(p, g2, e0, e1, oh0, oh1, me_p, ce_p, n_tokens, n_blocks,
                 capacity, trash_row):
    tok3 = lambda dt: jax.ShapeDtypeStruct((n_blocks, 1, _BT), dt)
    out_shapes = (
        tok3(jnp.int32), tok3(jnp.int32), tok3(jnp.int32), tok3(jnp.int32),
        tok3(jnp.float32), tok3(jnp.float32),
        jax.ShapeDtypeStruct((8, 128), jnp.float32),
    )
    blk3 = lambda: pl.BlockSpec((1, 1, _BT), lambda i: (i, 0, 0))
    full = lambda shape: pl.BlockSpec(shape, lambda i: tuple(0 for _ in shape))
    body = functools.partial(
        _routing_body, n_tokens=n_tokens, n_blocks=n_blocks,
        capacity=capacity, trash_row=trash_row)
    return pl.pallas_call(
        body,
        grid=(n_blocks,),
        in_specs=[
            full((1, n_tokens)), full((1, n_tokens)),
            full((1, n_tokens)), full((1, n_tokens)),
            full((n_tokens, E)), full((n_tokens, E)),
            full((n_blocks, 1, E)), full((n_blocks, 1, E)),
        ],
        out_specs=(
            blk3(), blk3(), blk3(), blk3(), blk3(), blk3(),
            pl.BlockSpec((8, 128), lambda i: (0, 0)),
        ),
        out_shape=out_shapes,
    )(p, g2, e0, e1, oh0, oh1, me_p, ce_p)


def _run_ffn(buf, w1, b1, w2, b2, capacity, n_rows):
    d_model = w1.shape[1]
    d_ff = w1.shape[2]
    n_h = d_ff // 1024
    return pl.pallas_call(
        _ffn_body,
        grid=(E, n_h),
        in_specs=[
            pl.BlockSpec((capacity, d_model), lambda e, h: (e, 0)),
            pl.BlockSpec((1, d_model, 1024), lambda e, h: (e, 0, h)),
            pl.BlockSpec((E, d_ff), lambda e, h: (0, 0)),
            pl.BlockSpec((1, 1024, d_model), lambda e, h: (e, h, 0)),
            pl.BlockSpec((E, d_model), lambda e, h: (0, 0)),
        ],
        out_specs=pl.BlockSpec((capacity, d_model), lambda e, h: (e, 0)),
        out_shape=jax.ShapeDtypeStruct((n_rows, d_model), jnp.float32),
    )(buf, w1, b1, w2, b2)


def _run_dispatch_sc(xf, ss0, ss1, n_rows):
    n, d = xf.shape
    per_w = n // _NW
    chunk = 64
    n_chunks = per_w // chunk
    mesh = plsc.VectorSubcoreMesh(core_axis_name="c", subcore_axis_name="s")

    @functools.partial(
        pl.kernel,
        out_type=jax.ShapeDtypeStruct((n_rows, d), jnp.float32),
        mesh=mesh,
        scratch_types=[
            pltpu.VMEM((chunk, d), jnp.float32),
            pltpu.VMEM((chunk,), jnp.int32),
            pltpu.VMEM((chunk,), jnp.int32),
            pltpu.SemaphoreType.DMA,
            pltpu.SemaphoreType.DMA,
        ],
    )
    def body(xf_hbm, ss0_hbm, ss1_hbm, buf_hbm, rows_v, idx0_v, idx1_v,
             sem0, sem1):
        wid = lax.axis_index("s") * _SC_CORES + lax.axis_index("c")
        for j in range(n_chunks):
            base = wid * per_w + j * chunk
            pltpu.sync_copy(xf_hbm.at[pl.ds(base, chunk)], rows_v)
            pltpu.sync_copy(ss0_hbm.at[pl.ds(base, chunk)], idx0_v)
            pltpu.sync_copy(ss1_hbm.at[pl.ds(base, chunk)], idx1_v)
            c0 = pltpu.async_copy(rows_v, buf_hbm.at[idx0_v], sem0)
            c1 = pltpu.async_copy(rows_v, buf_hbm.at[idx1_v], sem1)
            c0.wait()
            c1.wait()

    return body(xf, ss0, ss1)


def _lane_splat(vec16, i):
    idxv = jnp.full((16, 1), i, jnp.int32)
    return lax.gather(
        vec16, idxv,
        lax.GatherDimensionNumbers(offset_dims=(), collapsed_slice_dims=(0,),
                                   start_index_map=(0,)),
        (1,), mode=lax.GatherScatterMode.PROMISE_IN_BOUNDS)


def _run_combine_sc(out_rows, gs0, gs1, w0, w1, n, d):
    per_w = n // _NW
    chunk = 32
    n_chunks = per_w // chunk
    mesh = plsc.VectorSubcoreMesh(core_axis_name="c", subcore_axis_name="s")

    @functools.partial(
        pl.kernel,
        out_type=jax.ShapeDtypeStruct((n, d), jnp.float32),
        mesh=mesh,
        scratch_types=[
            pltpu.VMEM((chunk, d), jnp.float32),
            pltpu.VMEM((chunk, d), jnp.float32),
            pltpu.VMEM((chunk, d), jnp.float32),
            pltpu.VMEM((chunk,), jnp.int32),
            pltpu.VMEM((chunk,), jnp.int32),
            pltpu.VMEM((chunk,), jnp.float32),
            pltpu.VMEM((chunk,), jnp.float32),
            pltpu.SemaphoreType.DMA,
            pltpu.SemaphoreType.DMA,
        ],
    )
    def body(out_hbm, gs0_hbm, gs1_hbm, w0_hbm, w1_hbm, y_hbm,
             r0_v, r1_v, y_v, i0_v, i1_v, w0_v, w1_v, s0, s1):
        wid = lax.axis_index("s") * _SC_CORES + lax.axis_index("c")
        for j in range(n_chunks):
            base = wid * per_w + j * chunk
            pltpu.sync_copy(gs0_hbm.at[pl.ds(base, chunk)], i0_v)
            pltpu.sync_copy(gs1_hbm.at[pl.ds(base, chunk)], i1_v)
            pltpu.sync_copy(w0_hbm.at[pl.ds(base, chunk)], w0_v)
            pltpu.sync_copy(w1_hbm.at[pl.ds(base, chunk)], w1_v)
            c0 = pltpu.async_copy(out_hbm.at[i0_v], r0_v, s0)
            c1 = pltpu.async_copy(out_hbm.at[i1_v], r1_v, s1)
            c0.wait()
            c1.wait()

            for g in range(chunk // 16):
                wv0 = w0_v[pl.ds(g * 16, 16)]
                wv1 = w1_v[pl.ds(g * 16, 16)]

                def row_body(r16, carry, wv0=wv0, wv1=wv1, g=g):
                    w0s = _lane_splat(wv0, r16)
                    w1s = _lane_splat(wv1, r16)
                    r = g * 16 + r16
                    for cc in range(d // 16):
                        a = r0_v[r, pl.ds(cc * 16, 16)]
                        b = r1_v[r, pl.ds(cc * 16, 16)]
                        y_v[r, pl.ds(cc * 16, 16)] = a * w0s + b * w1s
                    return carry

                lax.fori_loop(0, 16, row_body, 0)
            pltpu.sync_copy(y_v, y_hbm.at[pl.ds(base, chunk)])

    return body(out_rows, gs0, gs1, w0, w1)


def kernel(x, wg, w1, b1, w2, b2):
    B, S, D = x.shape
    N = B * S
    n_blocks = N // _BT
    capacity = int(CAP_F * K * N / E)
    n_rows = (E + 1) * capacity
    trash_row = E * capacity
    xf = x.reshape(N, D)

    p3, g23, e03, e13, oh0, oh1, me_p, ce_p = _run_gating(xf, wg, n_blocks)
    p = p3.reshape(1, N)
    g2 = g23.reshape(1, N)
    e0 = e03.reshape(1, N)
    e1 = e13.reshape(1, N)

    (ss03, ss13, gs03, gs13, w03, w13, laux) = _run_routing(
        p, g2, e0, e1, oh0, oh1, me_p, ce_p, N, n_blocks, capacity, trash_row)
    ss0 = ss03.reshape(N)
    ss1 = ss13.reshape(N)
    gs0 = gs03.reshape(N)
    gs1 = gs13.reshape(N)
    w0 = w03.reshape(N)
    w1v = w13.reshape(N)

    buf = _run_dispatch_sc(xf, ss0, ss1, n_rows)
    out = _run_ffn(buf, w1, b1, w2, b2, capacity, n_rows)
    y = _run_combine_sc(out, gs0, gs1, w0, w1v, N, D)
    return y.reshape(B, S, D), laux[0, 0]


# speedup vs baseline: 2.8027x; 1.1098x over previous
import functools

import jax
import jax.numpy as jnp
from jax import lax
from jax.experimental import pallas as pl
from jax.experimental.pallas import tpu as pltpu
from jax.experimental.pallas import tpu_sc as plsc

_SC_CORES = 2
_SC_SUBCORES = 16
_NW = _SC_CORES * _SC_SUBCORES

E = 16
K = 2
CAP_F = 1.25
AUX_W = 0.01

_BT = 512


def _gating_body(x_ref, wg_ref, p_ref, g2_ref, e0_ref, e1_ref, ohc_ref,
                 me_ref, ce_ref):
    logits = jnp.dot(x_ref[...], wg_ref[...], preferred_element_type=jnp.float32)
    m = jnp.max(logits, axis=1, keepdims=True)
    ex = jnp.exp(logits - m)
    gates = ex / jnp.sum(ex, axis=1, keepdims=True)
    iota = jax.lax.broadcasted_iota(jnp.int32, gates.shape, 1)
    m1 = jnp.max(gates, axis=1, keepdims=True)
    is1 = gates == m1
    i1 = jnp.min(jnp.where(is1, iota, E), axis=1, keepdims=True)
    oh1h = (iota == i1)
    gm = jnp.where(oh1h, -jnp.inf, gates)
    m2 = jnp.max(gm, axis=1, keepdims=True)
    i2 = jnp.min(jnp.where(gm == m2, iota, E), axis=1, keepdims=True)
    oh2h = (iota == i2)
    p_ref[0, 0, :] = m1[:, 0]
    g2_ref[0, 0, :] = m2[:, 0]
    e0_ref[0, 0, :] = i1[:, 0]
    e1_ref[0, 0, :] = i2[:, 0]
    ohc_ref[:, :E] = oh1h.astype(jnp.bfloat16)
    ohc_ref[:, E:] = oh2h.astype(jnp.bfloat16)
    me_ref[0, 0, :] = jnp.sum(gates, axis=0)
    ce_ref[0, 0, :] = jnp.sum(oh1h.astype(jnp.float32), axis=0)


def _routing_body(p_ref, g2_ref, e0_ref, e1_ref, ohc_ref, me_ref, ce_ref,
                  sslot0_ref, sslot1_ref, gslot0_ref, gslot1_ref, w0_ref, w1_ref,
                  laux_ref, *, n_tokens, n_blocks, capacity, trash_row):
    i = pl.program_id(0)
    base_i = i * _BT
    pi = p_ref[0, pl.ds(base_i, _BT)].reshape(_BT, 1)
    ig = jax.lax.broadcasted_iota(jnp.int32, (_BT, 1), 0) + base_i

    def jstep(j, acc):
        base_j = j * _BT
        pj = p_ref[0, pl.ds(base_j, _BT)].reshape(1, _BT)
        jg = jax.lax.broadcasted_iota(jnp.int32, (1, _BT), 1) + base_j
        cmp = ((pj > pi) | ((pj == pi) & (jg < ig))).astype(jnp.bfloat16)
        ohj = ohc_ref[pl.ds(base_j, _BT), :]
        return acc + jnp.dot(cmp, ohj, preferred_element_type=jnp.float32)

    acc = jax.lax.fori_loop(
        0, n_blocks, jstep, jnp.zeros((_BT, 2 * E), jnp.float32))

    ohci = ohc_ref[pl.ds(base_i, _BT), :].astype(jnp.float32)
    oh0i = ohci[:, :E]
    oh1i = ohci[:, E:]
    rank0 = jnp.sum(acc[:, :E] * oh0i, axis=1)
    rank1 = jnp.sum(acc[:, E:] * oh1i, axis=1)
    cnt0 = jnp.sum(ohc_ref[:, :E].astype(jnp.float32), axis=0, keepdims=True)
    off1 = jnp.sum(cnt0 * oh1i, axis=1)
    pos0 = rank0.astype(jnp.int32)
    pos1 = (rank1 + off1).astype(jnp.int32)
    e0 = e0_ref[0, pl.ds(base_i, _BT)]
    e1 = e1_ref[0, pl.ds(base_i, _BT)]
    v0 = pos0 < capacity
    v1 = pos1 < capacity
    slot0 = e0 * capacity + jnp.minimum(pos0, capacity - 1)
    slot1 = e1 * capacity + jnp.minimum(pos1, capacity - 1)
    sslot0_ref[0, 0, :] = jnp.where(v0, slot0, trash_row)
    sslot1_ref[0, 0, :] = jnp.where(v1, slot1, trash_row)
    gslot0_ref[0, 0, :] = slot0
    gslot1_ref[0, 0, :] = slot1
    w0_ref[0, 0, :] = jnp.where(v0, p_ref[0, pl.ds(base_i, _BT)], 0.0)
    w1_ref[0, 0, :] = jnp.where(v1, g2_ref[0, pl.ds(base_i, _BT)], 0.0)

    @pl.when(i == 0)
    def _():
        me = jnp.sum(me_ref[:, 0, :], axis=0) / n_tokens
        ce = jnp.sum(ce_ref[:, 0, :], axis=0) / n_tokens
        laux = jnp.sum(me * ce) * E * AUX_W
        laux_ref[...] = jnp.full((8, 128), laux, jnp.float32)


def _ffn_body(x_ref, w1_ref, b1_ref, w2_ref, b2_ref, out_ref):
    e_id = pl.program_id(0)
    h_id = pl.program_id(1)
    xb = x_ref[...].astype(jnp.bfloat16)
    w1b = w1_ref[0].astype(jnp.bfloat16)
    b1v = b1_ref[pl.ds(e_id, 1), pl.ds(h_id * 1024, 1024)]
    h = jnp.dot(xb, w1b, preferred_element_type=jnp.float32) + b1v
    hg = 0.5 * h * (1.0 + jax.lax.erf(h * 0.7071067811865476))
    hb = hg.astype(jnp.bfloat16)
    w2b = w2_ref[0].astype(jnp.bfloat16)
    part = jnp.dot(hb, w2b, preferred_element_type=jnp.float32)

    @pl.when(h_id == 0)
    def _():
        out_ref[...] = part + b2_ref[pl.ds(e_id, 1), :]

    @pl.when(h_id != 0)
    def _():
        out_ref[...] = out_ref[...] + part


def _run_gating(xf, wg, n_blocks):
    n, d = xf.shape
    tok3 = lambda dt: jax.ShapeDtypeStruct((n_blocks, 1, _BT), dt)
    out_shapes = (
        tok3(jnp.float32), tok3(jnp.float32),
        tok3(jnp.int32), tok3(jnp.int32),
        jax.ShapeDtypeStruct((n, 2 * E), jnp.bfloat16),
        jax.ShapeDtypeStruct((n_blocks, 1, E), jnp.float32),
        jax.ShapeDtypeStruct((n_blocks, 1, E), jnp.float32),
    )
    blk3 = lambda: pl.BlockSpec((1, 1, _BT), lambda i: (i, 0, 0))
    return pl.pallas_call(
        _gating_body,
        grid=(n_blocks,),
        in_specs=[
            pl.BlockSpec((_BT, d), lambda i: (i, 0)),
            pl.BlockSpec(wg.shape, lambda i: (0, 0)),
        ],
        out_specs=(
            blk3(), blk3(), blk3(), blk3(),
            pl.BlockSpec((_BT, 2 * E), lambda i: (i, 0)),
            pl.BlockSpec((1, 1, E), lambda i: (i, 0, 0)),
            pl.BlockSpec((1, 1, E), lambda i: (i, 0, 0)),
        ),
        out_shape=out_shapes,
    )(xf, wg)


def _run_routing(p, g2, e0, e1, ohc, me_p, ce_p, n_tokens, n_blocks,
                 capacity, trash_row):
    tok3 = lambda dt: jax.ShapeDtypeStruct((n_blocks, 1, _BT), dt)
    out_shapes = (
        tok3(jnp.int32), tok3(jnp.int32), tok3(jnp.int32), tok3(jnp.int32),
        tok3(jnp.float32), tok3(jnp.float32),
        jax.ShapeDtypeStruct((8, 128), jnp.float32),
    )
    blk3 = lambda: pl.BlockSpec((1, 1, _BT), lambda i: (i, 0, 0))
    full = lambda shape: pl.BlockSpec(shape, lambda i: tuple(0 for _ in shape))
    body = functools.partial(
        _routing_body, n_tokens=n_tokens, n_blocks=n_blocks,
        capacity=capacity, trash_row=trash_row)
    return pl.pallas_call(
        body,
        grid=(n_blocks,),
        in_specs=[
            full((1, n_tokens)), full((1, n_tokens)),
            full((1, n_tokens)), full((1, n_tokens)),
            full((n_tokens, 2 * E)),
            full((n_blocks, 1, E)), full((n_blocks, 1, E)),
        ],
        out_specs=(
            blk3(), blk3(), blk3(), blk3(), blk3(), blk3(),
            pl.BlockSpec((8, 128), lambda i: (0, 0)),
        ),
        out_shape=out_shapes,
    )(p, g2, e0, e1, ohc, me_p, ce_p)


def _run_ffn(buf, w1, b1, w2, b2, capacity, n_rows):
    d_model = w1.shape[1]
    d_ff = w1.shape[2]
    n_h = d_ff // 1024
    return pl.pallas_call(
        _ffn_body,
        grid=(E, n_h),
        in_specs=[
            pl.BlockSpec((capacity, d_model), lambda e, h: (e, 0)),
            pl.BlockSpec((1, d_model, 1024), lambda e, h: (e, 0, h)),
            pl.BlockSpec((E, d_ff), lambda e, h: (0, 0)),
            pl.BlockSpec((1, 1024, d_model), lambda e, h: (e, h, 0)),
            pl.BlockSpec((E, d_model), lambda e, h: (0, 0)),
        ],
        out_specs=pl.BlockSpec((capacity, d_model), lambda e, h: (e, 0)),
        out_shape=jax.ShapeDtypeStruct((n_rows, d_model), jnp.float32),
    )(buf, w1, b1, w2, b2)


def _run_dispatch_sc(xf, ss0, ss1, n_rows):
    n, d = xf.shape
    per_w = n // _NW
    chunk = 64
    n_chunks = per_w // chunk
    mesh = plsc.VectorSubcoreMesh(core_axis_name="c", subcore_axis_name="s")

    @functools.partial(
        pl.kernel,
        out_type=jax.ShapeDtypeStruct((n_rows, d), jnp.float32),
        mesh=mesh,
        scratch_types=[
            pltpu.VMEM((chunk, d), jnp.float32),
            pltpu.VMEM((chunk,), jnp.int32),
            pltpu.VMEM((chunk,), jnp.int32),
            pltpu.SemaphoreType.DMA,
            pltpu.SemaphoreType.DMA,
        ],
    )
    def body(xf_hbm, ss0_hbm, ss1_hbm, buf_hbm, rows_v, idx0_v, idx1_v,
             sem0, sem1):
        wid = lax.axis_index("s") * _SC_CORES + lax.axis_index("c")
        for j in range(n_chunks):
            base = wid * per_w + j * chunk
            pltpu.sync_copy(xf_hbm.at[pl.ds(base, chunk)], rows_v)
            pltpu.sync_copy(ss0_hbm.at[pl.ds(base, chunk)], idx0_v)
            pltpu.sync_copy(ss1_hbm.at[pl.ds(base, chunk)], idx1_v)
            c0 = pltpu.async_copy(rows_v, buf_hbm.at[idx0_v], sem0)
            c1 = pltpu.async_copy(rows_v, buf_hbm.at[idx1_v], sem1)
            c0.wait()
            c1.wait()

    return body(xf, ss0, ss1)


def _lane_splat(vec16, i):
    idxv = jnp.full((16, 1), i, jnp.int32)
    return lax.gather(
        vec16, idxv,
        lax.GatherDimensionNumbers(offset_dims=(), collapsed_slice_dims=(0,),
                                   start_index_map=(0,)),
        (1,), mode=lax.GatherScatterMode.PROMISE_IN_BOUNDS)


def _run_combine_sc(out_rows, gs0, gs1, w0, w1, n, d):
    per_w = n // _NW
    chunk = 16
    n_chunks = per_w // chunk
    mesh = plsc.VectorSubcoreMesh(core_axis_name="c", subcore_axis_name="s")

    @functools.partial(
        pl.kernel,
        out_type=jax.ShapeDtypeStruct((n, d), jnp.float32),
        mesh=mesh,
        scratch_types=[
            pltpu.VMEM((2, chunk, d), jnp.float32),
            pltpu.VMEM((2, chunk, d), jnp.float32),
            pltpu.VMEM((2, chunk, d), jnp.float32),
            pltpu.VMEM((per_w,), jnp.int32),
            pltpu.VMEM((per_w,), jnp.int32),
            pltpu.VMEM((per_w,), jnp.float32),
            pltpu.VMEM((per_w,), jnp.float32),
            pltpu.SemaphoreType.DMA, pltpu.SemaphoreType.DMA,
            pltpu.SemaphoreType.DMA, pltpu.SemaphoreType.DMA,
            pltpu.SemaphoreType.DMA, pltpu.SemaphoreType.DMA,
        ],
    )
    def body(out_hbm, gs0_hbm, gs1_hbm, w0_hbm, w1_hbm, y_hbm,
             r0_v, r1_v, y_v, i0_v, i1_v, w0_v, w1_v,
             sg0a, sg0b, sg1a, sg1b, swa, swb):
        wid = lax.axis_index("s") * _SC_CORES + lax.axis_index("c")
        base = wid * per_w
        sg0 = [sg0a, sg0b]
        sg1 = [sg1a, sg1b]
        sw = [swa, swb]
        pltpu.sync_copy(gs0_hbm.at[pl.ds(base, per_w)], i0_v)
        pltpu.sync_copy(gs1_hbm.at[pl.ds(base, per_w)], i1_v)
        pltpu.sync_copy(w0_hbm.at[pl.ds(base, per_w)], w0_v)
        pltpu.sync_copy(w1_hbm.at[pl.ds(base, per_w)], w1_v)

        def gather_desc(j, b):
            d0 = pltpu.make_async_copy(
                out_hbm.at[i0_v.at[pl.ds(j * chunk, chunk)]], r0_v.at[b], sg0[b])
            d1 = pltpu.make_async_copy(
                out_hbm.at[i1_v.at[pl.ds(j * chunk, chunk)]], r1_v.at[b], sg1[b])
            return d0, d1

        def write_desc(j, b):
            return pltpu.make_async_copy(
                y_v.at[b], y_hbm.at[pl.ds(base + j * chunk, chunk)], sw[b])

        d0, d1 = gather_desc(0, 0)
        d0.start()
        d1.start()

        def step(jj, carry):
            for b in range(2):
                j = jj * 2 + b

                @pl.when(j < n_chunks - 1)
                def _(j=j, b=b):
                    n0, n1 = gather_desc(j + 1, 1 - b)
                    n0.start()
                    n1.start()

                @pl.when(j >= 2)
                def _(j=j, b=b):
                    write_desc(j - 2, b).wait()

                g0, g1 = gather_desc(j, b)
                g0.wait()
                g1.wait()
                wv0 = w0_v[pl.ds(j * chunk, chunk)]
                wv1 = w1_v[pl.ds(j * chunk, chunk)]

                def row_body(r16, carry2, wv0=wv0, wv1=wv1, b=b):
                    w0s = _lane_splat(wv0, r16)
                    w1s = _lane_splat(wv1, r16)
                    for cc in range(d // 16):
                        a = r0_v[b, r16, pl.ds(cc * 16, 16)]
                        c = r1_v[b, r16, pl.ds(cc * 16, 16)]
                        y_v[b, r16, pl.ds(cc * 16, 16)] = a * w0s + c * w1s
                    return carry2

                lax.fori_loop(0, chunk, row_body, 0)
                write_desc(j, b).start()
            return carry

        lax.fori_loop(0, n_chunks // 2, step, 0)
        write_desc(n_chunks - 2, 0).wait()
        write_desc(n_chunks - 1, 1).wait()

    return body(out_rows, gs0, gs1, w0, w1)


def kernel(x, wg, w1, b1, w2, b2):
    B, S, D = x.shape
    N = B * S
    n_blocks = N // _BT
    capacity = int(CAP_F * K * N / E)
    n_rows = (E + 1) * capacity
    trash_row = E * capacity
    xf = x.reshape(N, D)

    p3, g23, e03, e13, ohc, me_p, ce_p = _run_gating(xf, wg, n_blocks)
    p = p3.reshape(1, N)
    g2 = g23.reshape(1, N)
    e0 = e03.reshape(1, N)
    e1 = e13.reshape(1, N)

    (ss03, ss13, gs03, gs13, w03, w13, laux) = _run_routing(
        p, g2, e0, e1, ohc, me_p, ce_p, N, n_blocks, capacity, trash_row)
    ss0 = ss03.reshape(N)
    ss1 = ss13.reshape(N)
    gs0 = gs03.reshape(N)
    gs1 = gs13.reshape(N)
    w0 = w03.reshape(N)
    w1v = w13.reshape(N)

    buf = _run_dispatch_sc(xf, ss0, ss1, n_rows)
    out = _run_ffn(buf, w1, b1, w2, b2, capacity, n_rows)
    y = _run_combine_sc(out, gs0, gs1, w0, w1v, N, D)
    return y.reshape(B, S, D), laux[0, 0]


# speedup vs baseline: 2.8183x; 1.0056x over previous
import functools

import jax
import jax.numpy as jnp
from jax import lax
from jax.experimental import pallas as pl
from jax.experimental.pallas import tpu as pltpu
from jax.experimental.pallas import tpu_sc as plsc

_SC_CORES = 2
_SC_SUBCORES = 16
_NW = _SC_CORES * _SC_SUBCORES

E = 16
K = 2
CAP_F = 1.25
AUX_W = 0.01

_BT = 512


def _gating_body(x_ref, wg_ref, p_ref, g2_ref, e0_ref, e1_ref, ohc_ref,
                 me_ref, ce_ref):
    logits = jnp.dot(x_ref[...], wg_ref[...], preferred_element_type=jnp.float32)
    m = jnp.max(logits, axis=1, keepdims=True)
    ex = jnp.exp(logits - m)
    gates = ex / jnp.sum(ex, axis=1, keepdims=True)
    iota = jax.lax.broadcasted_iota(jnp.int32, gates.shape, 1)
    m1 = jnp.max(gates, axis=1, keepdims=True)
    is1 = gates == m1
    i1 = jnp.min(jnp.where(is1, iota, E), axis=1, keepdims=True)
    oh1h = (iota == i1)
    gm = jnp.where(oh1h, -jnp.inf, gates)
    m2 = jnp.max(gm, axis=1, keepdims=True)
    i2 = jnp.min(jnp.where(gm == m2, iota, E), axis=1, keepdims=True)
    oh2h = (iota == i2)
    p_ref[0, 0, :] = m1[:, 0]
    g2_ref[0, 0, :] = m2[:, 0]
    e0_ref[0, 0, :] = i1[:, 0]
    e1_ref[0, 0, :] = i2[:, 0]
    ohc_ref[:, :E] = oh1h.astype(jnp.bfloat16)
    ohc_ref[:, E:] = oh2h.astype(jnp.bfloat16)
    me_ref[0, 0, :] = jnp.sum(gates, axis=0)
    ce_ref[0, 0, :] = jnp.sum(oh1h.astype(jnp.float32), axis=0)


def _routing_body(p_ref, g2_ref, e0_ref, e1_ref, ohc_ref, me_ref, ce_ref,
                  sslot0_ref, sslot1_ref, gslot0_ref, gslot1_ref, w0_ref, w1_ref,
                  laux_ref, *, n_tokens, n_blocks, capacity, trash_row):
    i = pl.program_id(0)
    base_i = i * _BT
    pi = p_ref[0, pl.ds(base_i, _BT)].reshape(_BT, 1)
    ig = jax.lax.broadcasted_iota(jnp.int32, (_BT, 1), 0) + base_i

    def jstep(j, acc):
        base_j = j * _BT
        pj = p_ref[0, pl.ds(base_j, _BT)].reshape(1, _BT)
        jg = jax.lax.broadcasted_iota(jnp.int32, (1, _BT), 1) + base_j
        cmp = ((pj > pi) | ((pj == pi) & (jg < ig))).astype(jnp.bfloat16)
        ohj = ohc_ref[pl.ds(base_j, _BT), :]
        return acc + jnp.dot(cmp, ohj, preferred_element_type=jnp.float32)

    acc = jax.lax.fori_loop(
        0, n_blocks, jstep, jnp.zeros((_BT, 2 * E), jnp.float32))

    ohci = ohc_ref[pl.ds(base_i, _BT), :].astype(jnp.float32)
    oh0i = ohci[:, :E]
    oh1i = ohci[:, E:]
    rank0 = jnp.sum(acc[:, :E] * oh0i, axis=1)
    rank1 = jnp.sum(acc[:, E:] * oh1i, axis=1)
    cnt0 = jnp.sum(ohc_ref[:, :E].astype(jnp.float32), axis=0, keepdims=True)
    off1 = jnp.sum(cnt0 * oh1i, axis=1)
    pos0 = rank0.astype(jnp.int32)
    pos1 = (rank1 + off1).astype(jnp.int32)
    e0 = e0_ref[0, pl.ds(base_i, _BT)]
    e1 = e1_ref[0, pl.ds(base_i, _BT)]
    v0 = pos0 < capacity
    v1 = pos1 < capacity
    slot0 = e0 * capacity + jnp.minimum(pos0, capacity - 1)
    slot1 = e1 * capacity + jnp.minimum(pos1, capacity - 1)
    sslot0_ref[0, 0, :] = jnp.where(v0, slot0, trash_row)
    sslot1_ref[0, 0, :] = jnp.where(v1, slot1, trash_row)
    gslot0_ref[0, 0, :] = slot0
    gslot1_ref[0, 0, :] = slot1
    w0_ref[0, 0, :] = jnp.where(v0, p_ref[0, pl.ds(base_i, _BT)], 0.0)
    w1_ref[0, 0, :] = jnp.where(v1, g2_ref[0, pl.ds(base_i, _BT)], 0.0)

    @pl.when(i == 0)
    def _():
        me = jnp.sum(me_ref[:, 0, :], axis=0) / n_tokens
        ce = jnp.sum(ce_ref[:, 0, :], axis=0) / n_tokens
        laux = jnp.sum(me * ce) * E * AUX_W
        laux_ref[...] = jnp.full((8, 128), laux, jnp.float32)


_HB = 2048


def _ffn_body(x_ref, w1_ref, b1_ref, w2_ref, b2_ref, out_ref):
    e_id = pl.program_id(0)
    h_id = pl.program_id(1)
    xb = x_ref[...].astype(jnp.bfloat16)
    w1b = w1_ref[0].astype(jnp.bfloat16)
    b1v = b1_ref[pl.ds(e_id, 1), pl.ds(h_id * _HB, _HB)]
    h = jnp.dot(xb, w1b, preferred_element_type=jnp.float32) + b1v
    hg = 0.5 * h * (1.0 + jax.lax.erf(h * 0.7071067811865476))
    hb = hg.astype(jnp.bfloat16)
    w2b = w2_ref[0].astype(jnp.bfloat16)
    part = jnp.dot(hb, w2b, preferred_element_type=jnp.float32)

    @pl.when(h_id == 0)
    def _():
        out_ref[...] = part + b2_ref[pl.ds(e_id, 1), :]

    @pl.when(h_id != 0)
    def _():
        out_ref[...] = out_ref[...] + part


def _run_gating(xf, wg, n_blocks):
    n, d = xf.shape
    tok3 = lambda dt: jax.ShapeDtypeStruct((n_blocks, 1, _BT), dt)
    out_shapes = (
        tok3(jnp.float32), tok3(jnp.float32),
        tok3(jnp.int32), tok3(jnp.int32),
        jax.ShapeDtypeStruct((n, 2 * E), jnp.bfloat16),
        jax.ShapeDtypeStruct((n_blocks, 1, E), jnp.float32),
        jax.ShapeDtypeStruct((n_blocks, 1, E), jnp.float32),
    )
    blk3 = lambda: pl.BlockSpec((1, 1, _BT), lambda i: (i, 0, 0))
    return pl.pallas_call(
        _gating_body,
        grid=(n_blocks,),
        in_specs=[
            pl.BlockSpec((_BT, d), lambda i: (i, 0)),
            pl.BlockSpec(wg.shape, lambda i: (0, 0)),
        ],
        out_specs=(
            blk3(), blk3(), blk3(), blk3(),
            pl.BlockSpec((_BT, 2 * E), lambda i: (i, 0)),
            pl.BlockSpec((1, 1, E), lambda i: (i, 0, 0)),
            pl.BlockSpec((1, 1, E), lambda i: (i, 0, 0)),
        ),
        out_shape=out_shapes,
    )(xf, wg)


def _run_routing(p, g2, e0, e1, ohc, me_p, ce_p, n_tokens, n_blocks,
                 capacity, trash_row):
    tok3 = lambda dt: jax.ShapeDtypeStruct((n_blocks, 1, _BT), dt)
    out_shapes = (
        tok3(jnp.int32), tok3(jnp.int32), tok3(jnp.int32), tok3(jnp.int32),
        tok3(jnp.float32), tok3(jnp.float32),
        jax.ShapeDtypeStruct((8, 128), jnp.float32),
    )
    blk3 = lambda: pl.BlockSpec((1, 1, _BT), lambda i: (i, 0, 0))
    full = lambda shape: pl.BlockSpec(shape, lambda i: tuple(0 for _ in shape))
    body = functools.partial(
        _routing_body, n_tokens=n_tokens, n_blocks=n_blocks,
        capacity=capacity, trash_row=trash_row)
    return pl.pallas_call(
        body,
        grid=(n_blocks,),
        in_specs=[
            full((1, n_tokens)), full((1, n_tokens)),
            full((1, n_tokens)), full((1, n_tokens)),
            full((n_tokens, 2 * E)),
            full((n_blocks, 1, E)), full((n_blocks, 1, E)),
        ],
        out_specs=(
            blk3(), blk3(), blk3(), blk3(), blk3(), blk3(),
            pl.BlockSpec((8, 128), lambda i: (0, 0)),
        ),
        out_shape=out_shapes,
    )(p, g2, e0, e1, ohc, me_p, ce_p)


def _run_ffn(buf, w1, b1, w2, b2, capacity, n_rows):
    d_model = w1.shape[1]
    d_ff = w1.shape[2]
    n_h = d_ff // _HB
    return pl.pallas_call(
        _ffn_body,
        grid=(E, n_h),
        in_specs=[
            pl.BlockSpec((capacity, d_model), lambda e, h: (e, 0)),
            pl.BlockSpec((1, d_model, _HB), lambda e, h: (e, 0, h)),
            pl.BlockSpec((E, d_ff), lambda e, h: (0, 0)),
            pl.BlockSpec((1, _HB, d_model), lambda e, h: (e, h, 0)),
            pl.BlockSpec((E, d_model), lambda e, h: (0, 0)),
        ],
        out_specs=pl.BlockSpec((capacity, d_model), lambda e, h: (e, 0)),
        out_shape=jax.ShapeDtypeStruct((n_rows, d_model), jnp.float32),
        compiler_params=pltpu.CompilerParams(
            vmem_limit_bytes=110 * 1024 * 1024),
    )(buf, w1, b1, w2, b2)


def _run_dispatch_sc(xf, ss0, ss1, n_rows):
    n, d = xf.shape
    per_w = n // _NW
    chunk = 64
    n_chunks = per_w // chunk
    mesh = plsc.VectorSubcoreMesh(core_axis_name="c", subcore_axis_name="s")

    @functools.partial(
        pl.kernel,
        out_type=jax.ShapeDtypeStruct((n_rows, d), jnp.float32),
        mesh=mesh,
        scratch_types=[
            pltpu.VMEM((chunk, d), jnp.float32),
            pltpu.VMEM((chunk,), jnp.int32),
            pltpu.VMEM((chunk,), jnp.int32),
            pltpu.SemaphoreType.DMA,
            pltpu.SemaphoreType.DMA,
        ],
    )
    def body(xf_hbm, ss0_hbm, ss1_hbm, buf_hbm, rows_v, idx0_v, idx1_v,
             sem0, sem1):
        wid = lax.axis_index("s") * _SC_CORES + lax.axis_index("c")
        for j in range(n_chunks):
            base = wid * per_w + j * chunk
            pltpu.sync_copy(xf_hbm.at[pl.ds(base, chunk)], rows_v)
            pltpu.sync_copy(ss0_hbm.at[pl.ds(base, chunk)], idx0_v)
            pltpu.sync_copy(ss1_hbm.at[pl.ds(base, chunk)], idx1_v)
            c0 = pltpu.async_copy(rows_v, buf_hbm.at[idx0_v], sem0)
            c1 = pltpu.async_copy(rows_v, buf_hbm.at[idx1_v], sem1)
            c0.wait()
            c1.wait()

    return body(xf, ss0, ss1)


def _lane_splat(vec16, i):
    idxv = jnp.full((16, 1), i, jnp.int32)
    return lax.gather(
        vec16, idxv,
        lax.GatherDimensionNumbers(offset_dims=(), collapsed_slice_dims=(0,),
                                   start_index_map=(0,)),
        (1,), mode=lax.GatherScatterMode.PROMISE_IN_BOUNDS)


def _run_combine_sc(out_rows, gs0, gs1, w0, w1, n, d):
    per_w = n // _NW
    chunk = 16
    n_chunks = per_w // chunk
    mesh = plsc.VectorSubcoreMesh(core_axis_name="c", subcore_axis_name="s")

    @functools.partial(
        pl.kernel,
        out_type=jax.ShapeDtypeStruct((n, d), jnp.float32),
        mesh=mesh,
        scratch_types=[
            pltpu.VMEM((2, chunk, d), jnp.float32),
            pltpu.VMEM((2, chunk, d), jnp.float32),
            pltpu.VMEM((2, chunk, d), jnp.float32),
            pltpu.VMEM((per_w,), jnp.int32),
            pltpu.VMEM((per_w,), jnp.int32),
            pltpu.VMEM((per_w,), jnp.float32),
            pltpu.VMEM((per_w,), jnp.float32),
            pltpu.SemaphoreType.DMA, pltpu.SemaphoreType.DMA,
            pltpu.SemaphoreType.DMA, pltpu.SemaphoreType.DMA,
            pltpu.SemaphoreType.DMA, pltpu.SemaphoreType.DMA,
        ],
    )
    def body(out_hbm, gs0_hbm, gs1_hbm, w0_hbm, w1_hbm, y_hbm,
             r0_v, r1_v, y_v, i0_v, i1_v, w0_v, w1_v,
             sg0a, sg0b, sg1a, sg1b, swa, swb):
        wid = lax.axis_index("s") * _SC_CORES + lax.axis_index("c")
        base = wid * per_w
        sg0 = [sg0a, sg0b]
        sg1 = [sg1a, sg1b]
        sw = [swa, swb]
        pltpu.sync_copy(gs0_hbm.at[pl.ds(base, per_w)], i0_v)
        pltpu.sync_copy(gs1_hbm.at[pl.ds(base, per_w)], i1_v)
        pltpu.sync_copy(w0_hbm.at[pl.ds(base, per_w)], w0_v)
        pltpu.sync_copy(w1_hbm.at[pl.ds(base, per_w)], w1_v)

        def gather_desc(j, b):
            d0 = pltpu.make_async_copy(
                out_hbm.at[i0_v.at[pl.ds(j * chunk, chunk)]], r0_v.at[b], sg0[b])
            d1 = pltpu.make_async_copy(
                out_hbm.at[i1_v.at[pl.ds(j * chunk, chunk)]], r1_v.at[b], sg1[b])
            return d0, d1

        def write_desc(j, b):
            return pltpu.make_async_copy(
                y_v.at[b], y_hbm.at[pl.ds(base + j * chunk, chunk)], sw[b])

        d0, d1 = gather_desc(0, 0)
        d0.start()
        d1.start()

        def step(jj, carry):
            for b in range(2):
                j = jj * 2 + b

                @pl.when(j < n_chunks - 1)
                def _(j=j, b=b):
                    n0, n1 = gather_desc(j + 1, 1 - b)
                    n0.start()
                    n1.start()

                @pl.when(j >= 2)
                def _(j=j, b=b):
                    write_desc(j - 2, b).wait()

                g0, g1 = gather_desc(j, b)
                g0.wait()
                g1.wait()
                wv0 = w0_v[pl.ds(j * chunk, chunk)]
                wv1 = w1_v[pl.ds(j * chunk, chunk)]

                def row_body(r16, carry2, wv0=wv0, wv1=wv1, b=b):
                    w0s = _lane_splat(wv0, r16)
                    w1s = _lane_splat(wv1, r16)
                    for cc in range(d // 16):
                        a = r0_v[b, r16, pl.ds(cc * 16, 16)]
                        c = r1_v[b, r16, pl.ds(cc * 16, 16)]
                        y_v[b, r16, pl.ds(cc * 16, 16)] = a * w0s + c * w1s
                    return carry2

                lax.fori_loop(0, chunk, row_body, 0)
                write_desc(j, b).start()
            return carry

        lax.fori_loop(0, n_chunks // 2, step, 0)
        write_desc(n_chunks - 2, 0).wait()
        write_desc(n_chunks - 1, 1).wait()

    return body(out_rows, gs0, gs1, w0, w1)


def kernel(x, wg, w1, b1, w2, b2):
    B, S, D = x.shape
    N = B * S
    n_blocks = N // _BT
    capacity = int(CAP_F * K * N / E)
    n_rows = (E + 1) * capacity
    trash_row = E * capacity
    xf = x.reshape(N, D)

    p3, g23, e03, e13, ohc, me_p, ce_p = _run_gating(xf, wg, n_blocks)
    p = p3.reshape(1, N)
    g2 = g23.reshape(1, N)
    e0 = e03.reshape(1, N)
    e1 = e13.reshape(1, N)

    (ss03, ss13, gs03, gs13, w03, w13, laux) = _run_routing(
        p, g2, e0, e1, ohc, me_p, ce_p, N, n_blocks, capacity, trash_row)
    ss0 = ss03.reshape(N)
    ss1 = ss13.reshape(N)
    gs0 = gs03.reshape(N)
    gs1 = gs13.reshape(N)
    w0 = w03.reshape(N)
    w1v = w13.reshape(N)

    buf = _run_dispatch_sc(xf, ss0, ss1, n_rows)
    out = _run_ffn(buf, w1, b1, w2, b2, capacity, n_rows)
    y = _run_combine_sc(out, gs0, gs1, w0, w1v, N, D)
    return y.reshape(B, S, D), laux[0, 0]
